# baseline (speedup 1.0000x reference)
import jax
import jax.numpy as jnp
from jax import lax
from jax.experimental import pallas as pl
from jax.experimental.pallas import tpu as pltpu
from jax.experimental.pallas import tpu_sc as plsc

_N_DOCS = 100000
_BATCH = 4096
_LIST = 50
_NC = 2
_NS = 16
_NW = _NC * _NS
_ROWS_PER_W = _BATCH // _NW
_HALF = _ROWS_PER_W // 2
_GROUPS = _HALF // 16


def _cascade_body(x_hbm, table_hbm, out_hbm, idx_v, tab_v, out_v, sem_i, sem_t):
    wid = lax.axis_index("s") * _NC + lax.axis_index("c")
    base = wid * _ROWS_PER_W

    cp_i = pltpu.async_copy(x_hbm.at[pl.ds(base, _ROWS_PER_W)], idx_v, sem_i)
    cp_t = pltpu.async_copy(table_hbm, tab_v, sem_t)
    cp_i.wait()
    cp_t.wait()

    lane = lax.iota(jnp.int32, 16)
    zero16 = jnp.zeros((16,), jnp.int32)
    ones = jnp.ones((16,), jnp.float32)

    for h in range(2):
        def step(i, ps, h=h):
            col = zero16 + i
            new_ps = []
            for g in range(_GROUPS):
                row_l = lane + g * 16
                xi = plsc.load_gather(idx_v, [row_l + h * _HALF, col])
                v = plsc.load_gather(tab_v, [xi])
                r = 1.0 / (1.0 + jnp.exp(-v))
                o = ps[g] * r
                plsc.store_scatter(out_v, [row_l, col], o)
                new_ps.append(ps[g] - o)
            return tuple(new_ps)

        lax.fori_loop(0, _LIST, step, tuple(ones for _ in range(_GROUPS)))
        pltpu.sync_copy(out_v, out_hbm.at[pl.ds(base + h * _HALF, _HALF)])


def kernel(x, table):
    tf = table.reshape(_N_DOCS)
    mesh = plsc.VectorSubcoreMesh(core_axis_name="c", subcore_axis_name="s")
    return pl.kernel(
        _cascade_body,
        out_type=jax.ShapeDtypeStruct((_BATCH, _LIST), jnp.float32),
        mesh=mesh,
        compiler_params=pltpu.CompilerParams(needs_layout_passes=False),
        scratch_types=[
            pltpu.VMEM((_ROWS_PER_W, _LIST), jnp.int32),
            pltpu.VMEM((_N_DOCS,), jnp.float32),
            pltpu.VMEM((_HALF, _LIST), jnp.float32),
            pltpu.SemaphoreType.DMA,
            pltpu.SemaphoreType.DMA,
        ],
    )(x, tf)


# speedup vs baseline: 1.0814x; 1.0814x over previous
import jax
import jax.numpy as jnp
from jax import lax
from jax.experimental import pallas as pl
from jax.experimental.pallas import tpu as pltpu
from jax.experimental.pallas import tpu_sc as plsc

_N_DOCS = 100000
_BATCH = 4096
_LIST = 50
_NC = 2
_NS = 16
_NW = _NC * _NS
_ROWS_PER_W = _BATCH // _NW
_ELEMS_PER_W = _ROWS_PER_W * _LIST
_GROUPS = _ROWS_PER_W // 16


def _cascade_body(x_hbm, table_hbm, out_hbm, idx_v, tab_v, out_v, sem_i, sem_t):
    wid = lax.axis_index("s") * _NC + lax.axis_index("c")
    base = wid * _ELEMS_PER_W

    with jax.named_scope("stage"):
        cp_i = pltpu.async_copy(x_hbm.at[pl.ds(base, _ELEMS_PER_W)], idx_v, sem_i)
        cp_t = pltpu.async_copy(table_hbm, tab_v, sem_t)
        cp_i.wait()
        cp_t.wait()

    lane50 = lax.iota(jnp.int32, 16) * _LIST
    ones = jnp.ones((16,), jnp.float32)

    with jax.named_scope("casc"):
        def step(i, ps):
            new_ps = []
            for g in range(_GROUPS):
                lidx = lane50 + (g * 16 * _LIST + i)
                xi = plsc.load_gather(idx_v, [lidx])
                v = plsc.load_gather(tab_v, [xi])
                r = 1.0 / (1.0 + jnp.exp(-v))
                o = ps[g] * r
                plsc.store_scatter(out_v, [lidx], o)
                new_ps.append(ps[g] - o)
            return tuple(new_ps)

        lax.fori_loop(0, _LIST, step, tuple(ones for _ in range(_GROUPS)))

    with jax.named_scope("wb"):
        pltpu.sync_copy(out_v, out_hbm.at[pl.ds(base, _ELEMS_PER_W)])


def kernel(x, table):
    xf = x.reshape(_BATCH * _LIST)
    tf = table.reshape(_N_DOCS)
    mesh = plsc.VectorSubcoreMesh(core_axis_name="c", subcore_axis_name="s")
    out = pl.kernel(
        _cascade_body,
        out_type=jax.ShapeDtypeStruct((_BATCH * _LIST,), jnp.float32),
        mesh=mesh,
        compiler_params=pltpu.CompilerParams(needs_layout_passes=False),
        scratch_types=[
            pltpu.VMEM((_ELEMS_PER_W,), jnp.int32),
            pltpu.VMEM((_N_DOCS,), jnp.float32),
            pltpu.VMEM((_ELEMS_PER_W,), jnp.float32),
            pltpu.SemaphoreType.DMA,
            pltpu.SemaphoreType.DMA,
        ],
    )(xf, tf)
    return out.reshape(_BATCH, _LIST)


# speedup vs baseline: 1.3174x; 1.2182x over previous
import jax
import jax.numpy as jnp
from jax import lax
from jax.experimental import pallas as pl
from jax.experimental.pallas import tpu as pltpu
from jax.experimental.pallas import tpu_sc as plsc

_N_DOCS = 100000
_BATCH = 4096
_LIST = 50
_NC = 2
_NS = 16
_NW = _NC * _NS
_ROWS_PER_W = _BATCH // _NW
_ELEMS_PER_W = _ROWS_PER_W * _LIST
_GROUPS = _ROWS_PER_W // 16


def _cascade_body(x_hbm, table_hbm, out_hbm,
                  idx_v, idxt_v, tab_v, out_v, sem_i, sem_t):
    wid = lax.axis_index("s") * _NC + lax.axis_index("c")
    base = wid * _ELEMS_PER_W

    cp_i = pltpu.async_copy(x_hbm.at[pl.ds(base, _ELEMS_PER_W)], idx_v, sem_i)
    cp_t = pltpu.async_copy(table_hbm, tab_v, sem_t)
    cp_i.wait()

    lane50 = lax.iota(jnp.int32, 16) * _LIST
    ones = jnp.ones((16,), jnp.float32)

    with jax.named_scope("tr"):
        @plsc.parallel_loop(0, _LIST)
        def _tr(i):
            for g in range(_GROUPS):
                xi = plsc.load_gather(idx_v, [lane50 + (g * 16 * _LIST + i)])
                idxt_v[pl.ds(i * _ROWS_PER_W + g * 16, 16)] = xi

    with jax.named_scope("stage"):
        cp_t.wait()

    with jax.named_scope("casc"):
        @plsc.parallel_loop(0, _LIST, carry=tuple(ones for _ in range(_GROUPS)))
        def _casc(i, ps):
            new_ps = []
            for g in range(_GROUPS):
                xi = idxt_v[pl.ds(i * _ROWS_PER_W + g * 16, 16)]
                v = plsc.load_gather(tab_v, [xi])
                r = 1.0 / (1.0 + jnp.exp(-v))
                o = ps[g] * r
                plsc.store_scatter(out_v, [lane50 + (g * 16 * _LIST + i)], o)
                new_ps.append(ps[g] - o)
            return tuple(new_ps)

    with jax.named_scope("wb"):
        pltpu.sync_copy(out_v, out_hbm.at[pl.ds(base, _ELEMS_PER_W)])


def kernel(x, table):
    xf = x.reshape(_BATCH * _LIST)
    tf = table.reshape(_N_DOCS)
    mesh = plsc.VectorSubcoreMesh(core_axis_name="c", subcore_axis_name="s")
    out = pl.kernel(
        _cascade_body,
        out_type=jax.ShapeDtypeStruct((_BATCH * _LIST,), jnp.float32),
        mesh=mesh,
        compiler_params=pltpu.CompilerParams(needs_layout_passes=False),
        scratch_types=[
            pltpu.VMEM((_ELEMS_PER_W,), jnp.int32),
            pltpu.VMEM((_ELEMS_PER_W,), jnp.int32),
            pltpu.VMEM((_N_DOCS,), jnp.float32),
            pltpu.VMEM((_ELEMS_PER_W,), jnp.float32),
            pltpu.SemaphoreType.DMA,
            pltpu.SemaphoreType.DMA,
        ],
    )(xf, tf)
    return out.reshape(_BATCH, _LIST)


# speedup vs baseline: 1.5924x; 1.2088x over previous
import jax
import jax.numpy as jnp
from jax import lax
from jax.experimental import pallas as pl
from jax.experimental.pallas import tpu as pltpu
from jax.experimental.pallas import tpu_sc as plsc

_N_DOCS = 100000
_BATCH = 4096
_LIST = 50
_NC = 2
_NS = 16
_NW = _NC * _NS
_ROWS_PER_W = _BATCH // _NW
_ELEMS_PER_W = _ROWS_PER_W * _LIST
_GROUPS = _ROWS_PER_W // 16


def _cascade_body(x_hbm, table_hbm, out_hbm,
                  idx_v, idxt_v, vals_v, out_v, shared_tab,
                  sem_i, sem_t, sem_g):
    cid = lax.axis_index("c")
    sid = lax.axis_index("s")
    wid = sid * _NC + cid
    base = wid * _ELEMS_PER_W

    cp_i = pltpu.async_copy(x_hbm.at[pl.ds(base, _ELEMS_PER_W)], idx_v, sem_i)

    with jax.named_scope("spfill"):
        @pl.when(sid == 0)
        def _fill():
            pltpu.async_copy(table_hbm, shared_tab, sem_t).wait()

    cp_i.wait()

    lane50 = lax.iota(jnp.int32, 16) * _LIST
    ones = jnp.ones((16,), jnp.float32)

    with jax.named_scope("tr"):
        @plsc.parallel_loop(0, _LIST)
        def _tr(i):
            for g in range(_GROUPS):
                xi = plsc.load_gather(idx_v, [lane50 + (g * 16 * _LIST + i)])
                idxt_v[pl.ds(i * _ROWS_PER_W + g * 16, 16)] = xi

    with jax.named_scope("bar"):
        plsc.subcore_barrier()

    with jax.named_scope("gather"):
        pltpu.async_copy(shared_tab.at[idxt_v], vals_v, sem_g).wait()

    with jax.named_scope("casc"):
        @plsc.parallel_loop(0, _LIST, carry=tuple(ones for _ in range(_GROUPS)))
        def _casc(i, ps):
            new_ps = []
            for g in range(_GROUPS):
                v = vals_v[pl.ds(i * _ROWS_PER_W + g * 16, 16)]
                r = 1.0 / (1.0 + jnp.exp(-v))
                o = ps[g] * r
                plsc.store_scatter(out_v, [lane50 + (g * 16 * _LIST + i)], o)
                new_ps.append(ps[g] - o)
            return tuple(new_ps)

    with jax.named_scope("wb"):
        pltpu.sync_copy(out_v, out_hbm.at[pl.ds(base, _ELEMS_PER_W)])


def kernel(x, table):
    xf = x.reshape(_BATCH * _LIST)
    tf = table.reshape(_N_DOCS)
    mesh = plsc.VectorSubcoreMesh(core_axis_name="c", subcore_axis_name="s")
    out = pl.kernel(
        _cascade_body,
        out_type=jax.ShapeDtypeStruct((_BATCH * _LIST,), jnp.float32),
        mesh=mesh,
        compiler_params=pltpu.CompilerParams(needs_layout_passes=False),
        scratch_types=[
            pltpu.VMEM((_ELEMS_PER_W,), jnp.int32),
            pltpu.VMEM((_ELEMS_PER_W,), jnp.int32),
            pltpu.VMEM((_ELEMS_PER_W,), jnp.float32),
            pltpu.VMEM((_ELEMS_PER_W,), jnp.float32),
            pltpu.VMEM_SHARED((_N_DOCS,), jnp.float32),
            pltpu.SemaphoreType.DMA,
            pltpu.SemaphoreType.DMA,
            pltpu.SemaphoreType.DMA,
        ],
    )(xf, tf)
    return out.reshape(_BATCH, _LIST)
